# baseline (speedup 1.0000x reference)
import functools

import jax
import jax.numpy as jnp
from jax import lax
from jax.experimental import pallas as pl
from jax.experimental.pallas import tpu as pltpu
from jax.experimental.pallas import tpu_sc as plsc

_VOCAB = 38462 * 26
_D = 16
_B = 16384
_F = 26

_INFO = plsc.get_sparse_core_info()
_NC = _INFO.num_cores
_NS = _INFO.num_subcores
_NW = _NC * _NS
_VPAD = 1000064
_NBLK = _VPAD // 128


@functools.partial(
    pl.kernel,
    mesh=plsc.VectorSubcoreMesh(core_axis_name="c", subcore_axis_name="s"),
    out_type=jax.ShapeDtypeStruct((_NBLK * 2048,), jnp.float32),
    scratch_types=[
        pltpu.VMEM((2, 2, 8, 128), jnp.float32),
        pltpu.VMEM((2, 2048), jnp.float32),
        pltpu.SemaphoreType.DMA((2,)),
        pltpu.SemaphoreType.DMA((2,)),
    ],
    compiler_params=pltpu.CompilerParams(
        use_tc_tiling_on_sc=True, needs_layout_passes=False),
)
def _sc_retile(tT_hbm, flat8_hbm, vin, vout, gsem, wsem):
    wid = lax.axis_index("s") * _NC + lax.axis_index("c")
    i16x16 = lax.iota(jnp.int32, 16) * 16
    niter = (_NBLK + _NW - 1) // _NW

    def fire(i, p):
        j = wid + i * _NW

        @pl.when(j < _NBLK)
        def _():
            pltpu.async_copy(
                tT_hbm.at[pl.ds(0, 8), pl.ds(j * 128, 128)], vin.at[p, 0],
                gsem.at[p])
            pltpu.async_copy(
                tT_hbm.at[pl.ds(8, 8), pl.ds(j * 128, 128)], vin.at[p, 1],
                gsem.at[p])

    fire(0, 0)

    def _half(i, p):
        j = wid + i * _NW
        fire(i + 1, 1 - p)

        @pl.when(j < _NBLK)
        def _():
            pltpu.make_async_copy(
                tT_hbm.at[pl.ds(0, 8), pl.ds(0, 128)], vin.at[p, 0],
                gsem.at[p]).wait()
            pltpu.make_async_copy(
                tT_hbm.at[pl.ds(0, 8), pl.ds(0, 128)], vin.at[p, 1],
                gsem.at[p]).wait()
            @pl.when(i >= 2)
            def _():
                pltpu.make_async_copy(
                    vout.at[p], flat8_hbm.at[pl.ds(0, 2048)], wsem.at[p]).wait()

            psel = jnp.full((16,), p, jnp.int32)
            for i_t in range(2):
                for es in range(8):
                    e = 8 * i_t + es
                    vals = [vin[p, i_t, es, pl.ds(16 * c, 16)]
                            for c in range(8)]
                    for c in range(8):
                        plsc.store_scatter(
                            vout, [psel, i16x16 + (e + 256 * c)], vals[c])

            pltpu.async_copy(
                vout.at[p], flat8_hbm.at[pl.ds(2048 * j, 2048)], wsem.at[p])

    def step(i, carry):
        for u in range(2):
            @pl.when(lax.rem(i, 2) == u)
            def _(u=u):
                _half(i, u)
        return carry

    lax.fori_loop(0, niter, step, 0)
    for p in range(2):
        pltpu.make_async_copy(
            vout.at[p], flat8_hbm.at[pl.ds(0, 2048)], wsem.at[p]).wait()
_BC = 1024
_NT = _F * (_B // _BC)
_TPW = _NT // _NW
_SUB = 128
_NSUB = _BC // _SUB


@functools.partial(
    pl.kernel,
    mesh=plsc.VectorSubcoreMesh(core_axis_name="c", subcore_axis_name="s"),
    out_type=jax.ShapeDtypeStruct((_F, _D, _B), jnp.float32),
    scratch_types=[
        pltpu.VMEM((_BC,), jnp.int32),
        pltpu.VMEM((2, _SUB, _D), jnp.float32),
        pltpu.VMEM((_D, _BC), jnp.float32),
        pltpu.SemaphoreType.DMA,
        pltpu.SemaphoreType.DMA,
        pltpu.SemaphoreType.DMA,
    ],
    compiler_params=pltpu.CompilerParams(
        use_tc_tiling_on_sc=False, needs_layout_passes=False),
)
def _sc_embed(xT_hbm, tab_hbm, outT_hbm, idx_v, rows_v, tbuf, gsem0, gsem1, wsem):
    wid = lax.axis_index("s") * _NC + lax.axis_index("c")
    iota16 = lax.iota(jnp.int32, 16)
    gsems = (gsem0, gsem1)

    def task(t_local, carry):
        t = wid * _TPW + t_local
        f = t // (_B // _BC)
        c = lax.rem(t, _B // _BC)

        pltpu.sync_copy(xT_hbm.at[f, pl.ds(c * _BC, _BC)], idx_v)

        descs = [None] * _NSUB
        descs[0] = pltpu.async_copy(
            tab_hbm.at[idx_v.at[pl.ds(0, _SUB)]], rows_v.at[0], gsems[0])

        @pl.when(t_local > 0)
        def _():
            pltpu.make_async_copy(
                tbuf, outT_hbm.at[0, :, pl.ds(0, _BC)], wsem).wait()

        for s in range(_NSUB):
            p = s % 2
            if s + 1 < _NSUB:
                descs[s + 1] = pltpu.async_copy(
                    tab_hbm.at[idx_v.at[pl.ds((s + 1) * _SUB, _SUB)]],
                    rows_v.at[1 - p], gsems[(s + 1) % 2])
            descs[s].wait()
            for j0 in range(0, _SUB, 8):
                vals = [rows_v[p, j0 + u, :] for u in range(8)]
                for u in range(8):
                    plsc.store_scatter(
                        tbuf,
                        [iota16, jnp.full((16,), s * _SUB + j0 + u, jnp.int32)],
                        vals[u])

        pltpu.async_copy(tbuf, outT_hbm.at[f, :, pl.ds(c * _BC, _BC)], wsem)
        return carry

    lax.fori_loop(0, _TPW, task, 0)
    pltpu.make_async_copy(tbuf, outT_hbm.at[0, :, pl.ds(0, _BC)], wsem).wait()


def kernel(x, table):
    flat8 = _sc_retile(table.T)
    tab = flat8.reshape(_VPAD, _D)
    xT = x.T.astype(jnp.int32)
    outT = _sc_embed(xT, tab)
    return outT.transpose(2, 0, 1)


# speedup vs baseline: 1.0977x; 1.0977x over previous
import functools

import jax
import jax.numpy as jnp
from jax import lax
from jax.experimental import pallas as pl
from jax.experimental.pallas import tpu as pltpu
from jax.experimental.pallas import tpu_sc as plsc

_VOCAB = 38462 * 26
_D = 16
_B = 16384
_F = 26

_INFO = plsc.get_sparse_core_info()
_NC = _INFO.num_cores
_NS = _INFO.num_subcores
_NW = _NC * _NS
_VPAD = 1000064
_NBLK = _VPAD // 128


@functools.partial(
    pl.kernel,
    mesh=plsc.VectorSubcoreMesh(core_axis_name="c", subcore_axis_name="s"),
    out_type=jax.ShapeDtypeStruct((_NBLK * 2048,), jnp.float32),
    scratch_types=[
        pltpu.VMEM((2, 2, 8, 128), jnp.float32),
        pltpu.VMEM((2, 2048), jnp.float32),
        pltpu.SemaphoreType.DMA((2,)),
        pltpu.SemaphoreType.DMA((2,)),
    ],
    compiler_params=pltpu.CompilerParams(
        use_tc_tiling_on_sc=True, needs_layout_passes=False),
)
def _sc_retile(tT_hbm, flat8_hbm, vin, vout, gsem, wsem):
    wid = lax.axis_index("s") * _NC + lax.axis_index("c")
    i16x16 = lax.iota(jnp.int32, 16) * 16
    niter = (_NBLK + _NW - 1) // _NW

    def fire(i, p):
        j = wid + i * _NW

        @pl.when(j < _NBLK)
        def _():
            pltpu.async_copy(
                tT_hbm.at[pl.ds(0, 8), pl.ds(j * 128, 128)], vin.at[p, 0],
                gsem.at[p])
            pltpu.async_copy(
                tT_hbm.at[pl.ds(8, 8), pl.ds(j * 128, 128)], vin.at[p, 1],
                gsem.at[p])

    fire(0, 0)

    def _half(i, p):
        j = wid + i * _NW
        fire(i + 1, 1 - p)

        @pl.when(j < _NBLK)
        def _():
            pltpu.make_async_copy(
                tT_hbm.at[pl.ds(0, 8), pl.ds(0, 128)], vin.at[p, 0],
                gsem.at[p]).wait()
            pltpu.make_async_copy(
                tT_hbm.at[pl.ds(0, 8), pl.ds(0, 128)], vin.at[p, 1],
                gsem.at[p]).wait()
            @pl.when(i >= 2)
            def _():
                pltpu.make_async_copy(
                    vout.at[p], flat8_hbm.at[pl.ds(0, 2048)], wsem.at[p]).wait()

            psel = jnp.full((16,), p, jnp.int32)
            for i_t in range(2):
                for es in range(8):
                    e = 8 * i_t + es
                    vals = [vin[p, i_t, es, pl.ds(16 * c, 16)]
                            for c in range(8)]
                    for c in range(8):
                        plsc.store_scatter(
                            vout, [psel, i16x16 + (e + 256 * c)], vals[c])

            pltpu.async_copy(
                vout.at[p], flat8_hbm.at[pl.ds(2048 * j, 2048)], wsem.at[p])

    def step(i, carry):
        for u in range(2):
            @pl.when(lax.rem(i, 2) == u)
            def _(u=u):
                _half(i, u)
        return carry

    lax.fori_loop(0, niter, step, 0)
    for p in range(2):
        pltpu.make_async_copy(
            vout.at[p], flat8_hbm.at[pl.ds(0, 2048)], wsem.at[p]).wait()
_BC = 1024
_NT = _F * (_B // _BC)
_TPW = _NT // _NW
_SUB = 128
_NSUB = _BC // _SUB


@functools.partial(
    pl.kernel,
    mesh=plsc.VectorSubcoreMesh(core_axis_name="c", subcore_axis_name="s"),
    out_type=jax.ShapeDtypeStruct((_F, _D, _B), jnp.float32),
    scratch_types=[
        pltpu.VMEM((_BC,), jnp.int32),
        pltpu.VMEM((2, _SUB, _D), jnp.float32),
        pltpu.VMEM((_D, _BC), jnp.float32),
        pltpu.SemaphoreType.DMA,
        pltpu.SemaphoreType.DMA,
        pltpu.SemaphoreType.DMA,
    ],
    compiler_params=pltpu.CompilerParams(
        use_tc_tiling_on_sc=False, needs_layout_passes=False),
)
def _sc_embed(xT_hbm, tab_hbm, outT_hbm, idx_v, rows_v, tbuf, gsem0, gsem1, wsem):
    wid = lax.axis_index("s") * _NC + lax.axis_index("c")
    iota16 = lax.iota(jnp.int32, 16)
    gsems = (gsem0, gsem1)

    def task(t_local, carry):
        t = wid * _TPW + t_local
        f = t // (_B // _BC)
        c = lax.rem(t, _B // _BC)

        pltpu.sync_copy(xT_hbm.at[f, pl.ds(c * _BC, _BC)], idx_v)

        descs = [None] * _NSUB
        descs[0] = pltpu.async_copy(
            tab_hbm.at[idx_v.at[pl.ds(0, _SUB)]], rows_v.at[0], gsems[0])

        @pl.when(t_local > 0)
        def _():
            pltpu.make_async_copy(
                tbuf, outT_hbm.at[0, :, pl.ds(0, _BC)], wsem).wait()

        for s in range(_NSUB):
            p = s % 2
            if s + 1 < _NSUB:
                descs[s + 1] = pltpu.async_copy(
                    tab_hbm.at[idx_v.at[pl.ds((s + 1) * _SUB, _SUB)]],
                    rows_v.at[1 - p], gsems[(s + 1) % 2])
            descs[s].wait()
            for g in range(_SUB // 16):
                ridx = iota16 + (g * 16)
                for e0 in range(0, _D, 8):
                    vals = [plsc.load_gather(
                        rows_v,
                        [jnp.full((16,), p, jnp.int32), ridx,
                         jnp.full((16,), e0 + u, jnp.int32)]) for u in range(8)]
                    for u in range(8):
                        tbuf[e0 + u, pl.ds(s * _SUB + g * 16, 16)] = vals[u]

        pltpu.async_copy(tbuf, outT_hbm.at[f, :, pl.ds(c * _BC, _BC)], wsem)
        return carry

    lax.fori_loop(0, _TPW, task, 0)
    pltpu.make_async_copy(tbuf, outT_hbm.at[0, :, pl.ds(0, _BC)], wsem).wait()


def kernel(x, table):
    flat8 = _sc_retile(table.T)
    tab = flat8.reshape(_VPAD, _D)
    xT = x.T.astype(jnp.int32)
    outT = _sc_embed(xT, tab)
    return outT.transpose(2, 0, 1)
